# baseline (speedup 1.0000x reference)
import jax
import jax.numpy as jnp
from jax import lax
from jax.experimental import pallas as pl
from jax.experimental.pallas import tpu as pltpu
from jax.experimental.pallas import tpu_sc as plsc

N = 10000
NP = 10240
E = 320000
D = 128
K = 8
NC = 2
NS = 16
L = 16
DH = D // NC
FW = DH // L
B = 128
TB = 160
TE = TB * B
EP = NS * TE
RPT = NP // NS
RC = 128
NRC = RPT // RC
NR = 4


def _body(xs, sd, wbh, dtr, omr, zro, zrd, out, wnb,
          agg_sh, deg_sh,
          isd0, isd1, isd2, isd3,
          wb0, wb1, wb2, wb3, wrow0, wrow1,
          rw0, rw1, rw2, rw3,
          uh, ua, dtv, omv,
          sg0, sg1, sg2, sg3, ss0, ss1, ss2, ss3,
          sw0, sw1, sw2, sw3, si0, si1, si2, si3):
    cid = lax.axis_index("c")
    sid = lax.axis_index("s")
    hbase = cid * NP
    ebase = sid * TE
    ibase = (cid * NS + sid) * TB * 2 * B

    isd = (isd0, isd1, isd2, isd3)
    wbr = (wb0, wb1, wb2, wb3)
    wrow = (wrow0, wrow1)
    rows = (rw0, rw1, rw2, rw3)
    sg = (sg0, sg1, sg2, sg3)
    ss = (ss0, ss1, ss2, ss3)
    sw = (sw0, sw1, sw2, sw3)
    si = (si0, si1, si2, si3)

    pltpu.sync_copy(dtr, dtv)
    pltpu.sync_copy(omr, omv)

    for ch in range(NRC):
        r0 = sid * RPT + ch * RC
        pltpu.sync_copy(zro, agg_sh.at[pl.ds(r0, RC)])
        pltpu.sync_copy(zrd, deg_sh.at[pl.ds(r0, RC)])
        pltpu.sync_copy(xs.at[pl.ds(hbase + r0, RC)], uh)
        pltpu.sync_copy(uh, out.at[pl.ds(hbase + r0, RC)])

    plsc.subcore_barrier()

    def _di_start(b1, r):
        pltpu.async_copy(sd.at[pl.ds(ibase + b1 * 2 * B, 2 * B)],
                         isd[r], si[r])

    def _di_wait(b1, r):
        pltpu.make_async_copy(sd.at[pl.ds(ibase + b1 * 2 * B, 2 * B)],
                              isd[r], si[r]).wait()

    def _srcx(r):
        return isd[r].at[pl.ds(0, B)]

    def _dstx(r):
        return isd[r].at[pl.ds(B, B)]

    def _w_start(b1, r):
        pltpu.async_copy(wbh.at[pl.ds(ebase + b1 * B, B)], wbr[r], sw[r])

    def _w_wait(b1, r):
        pltpu.make_async_copy(
            wbh.at[pl.ds(ebase + b1 * B, B)], wbr[r], sw[r]).wait()

    def _wn_start(b1, r):
        pltpu.async_copy(wnb.at[pl.ds(ebase + b1 * B, B)], wbr[r], sw[r])

    def _wn_wait(b1, r):
        pltpu.make_async_copy(
            wnb.at[pl.ds(ebase + b1 * B, B)], wbr[r], sw[r]).wait()

    _di_start(0, 0)
    _w_start(0, 0)

    def _degA(j, c):
        for p in (0, 1):
            i = 2 * j + p
            q = 1 - p

            @pl.when(i + 1 < TB)
            def _():
                @pl.when(i >= 1)
                def _():
                    pltpu.make_async_copy(
                        wbr[q], deg_sh.at[_dstx(q)], ss[q]).wait()
                _di_start(i + 1, q)
                _w_start(i + 1, q)

            _di_wait(i, p)
            _w_wait(i, p)
            pltpu.async_copy(wbr[p], deg_sh.at[_dstx(p)], ss[p], add=True)
        return c
    lax.fori_loop(0, TB // 2, _degA, 0)
    pltpu.make_async_copy(wbr[0], deg_sh.at[_dstx(0)], ss[0]).wait()
    pltpu.make_async_copy(wbr[1], deg_sh.at[_dstx(1)], ss[1]).wait()

    plsc.subcore_barrier()

    def _deg_start(p):
        pltpu.async_copy(deg_sh.at[_dstx(p)], wrow[p], sg[p])

    def _deg_wait(p):
        pltpu.make_async_copy(deg_sh.at[_dstx(p)], wrow[p], sg[p]).wait()

    _di_start(0, 0)
    _di_wait(0, 0)
    _w_start(0, 0)
    _deg_start(0)

    def _wnB(j, c):
        for p in (0, 1):
            i = 2 * j + p
            q = 1 - p

            @pl.when(i + 1 < TB)
            def _():
                @pl.when(i >= 1)
                def _():
                    pltpu.make_async_copy(
                        wbr[q], wnb.at[pl.ds(ebase + (i - 1) * B, B)],
                        sg[2 + q]).wait()
                _di_start(i + 1, q)
                _w_start(i + 1, q)
                _di_wait(i + 1, q)
                _deg_start(q)

            _w_wait(i, p)
            _deg_wait(p)

            def _edge(e2, c2):
                dv = dtv[...]
                for u in range(2):
                    e = e2 * 2 + u
                    wbr[p][e] = dv * wbr[p][e] / (wrow[p][e] + 1e-8)
                return c2
            lax.fori_loop(0, B // 2, _edge, 0)

            pltpu.async_copy(
                wbr[p], wnb.at[pl.ds(ebase + i * B, B)], sg[2 + p])
        return c
    lax.fori_loop(0, TB // 2, _wnB, 0)
    pltpu.make_async_copy(
        wbr[0], wnb.at[pl.ds(ebase + (TB - 2) * B, B)], sg[2]).wait()
    pltpu.make_async_copy(
        wbr[1], wnb.at[pl.ds(ebase + (TB - 1) * B, B)], sg[3]).wait()

    plsc.subcore_barrier()

    def _gather_start(b1, r):
        pltpu.async_copy(out.at[_srcx(r)], rows[r], sg[r])

    def _gather_wait(b1, r):
        pltpu.make_async_copy(out.at[_srcx(r)], rows[r], sg[r]).wait()

    def _scat_start(b1, r):
        pltpu.async_copy(rows[r], agg_sh.at[_dstx(r)], ss[r], add=True)

    def _scat_wait(b1, r):
        pltpu.make_async_copy(rows[r], agg_sh.at[_dstx(r)], ss[r]).wait()

    def _step(k, c):
        _di_start(0, 0)
        _di_wait(0, 0)
        _wn_start(0, 0)
        _gather_start(0, 0)
        _di_start(1, 1)

        def _quad(j, c2):
            for r in range(NR):
                i = NR * j + r
                r1 = (r + 1) % NR
                r2 = (r + 2) % NR

                @pl.when(i + 2 < TB)
                def _():
                    @pl.when(i >= 2)
                    def _():
                        _scat_wait(i - 2, r2)
                    _di_start(i + 2, r2)

                @pl.when(i + 1 < TB)
                def _():
                    _di_wait(i + 1, r1)
                    _wn_start(i + 1, r1)
                    _gather_start(i + 1, r1)

                _gather_wait(i, r)
                _wn_wait(i, r)

                def _edge(e4, c3):
                    for u in range(4):
                        e = e4 * 4 + u
                        wv = wbr[r][e]
                        for f in range(FW):
                            rows[r][e, f] = rows[r][e, f] * wv
                    return c3
                lax.fori_loop(0, B // 4, _edge, 0)

                _scat_start(i, r)
            return c2
        lax.fori_loop(0, TB // NR, _quad, 0)
        _scat_wait(TB - 4, 0)
        _scat_wait(TB - 3, 1)
        _scat_wait(TB - 2, 2)
        _scat_wait(TB - 1, 3)

        plsc.subcore_barrier()

        for ch in range(NRC):
            r0l = sid * RPT + ch * RC
            r0g = hbase + r0l
            pltpu.sync_copy(out.at[pl.ds(r0g, RC)], uh)
            pltpu.sync_copy(agg_sh.at[pl.ds(r0l, RC)], ua)

            def _upd(r2c, c2):
                om = omv[...]
                for u in range(2):
                    rr = r2c * 2 + u
                    for f in range(FW):
                        uh[rr, f] = om * uh[rr, f] + ua[rr, f]
                return c2
            lax.fori_loop(0, RC // 2, _upd, 0)

            pltpu.sync_copy(uh, out.at[pl.ds(r0g, RC)])
            pltpu.sync_copy(zro, agg_sh.at[pl.ds(r0l, RC)])

        plsc.subcore_barrier()
        return c
    lax.fori_loop(0, K, _step, 0)


@jax.jit
def _diffuse(xs, sd, w_b, dtv, omv, zro, zrd):
    mesh = plsc.VectorSubcoreMesh(core_axis_name="c", subcore_axis_name="s",
                                  num_cores=NC, num_subcores=NS)
    f = pl.kernel(
        _body,
        out_type=(
            jax.ShapeDtypeStruct((NC * NP, FW, L), jnp.float32),
            jax.ShapeDtypeStruct((EP, L), jnp.float32),
        ),
        mesh=mesh,
        scratch_types=(
            [pltpu.VMEM_SHARED((NP, FW, L), jnp.float32),
             pltpu.VMEM_SHARED((NP, L), jnp.float32)]
            + [pltpu.VMEM((2 * B,), jnp.int32)] * 4
            + [pltpu.VMEM((B, L), jnp.float32)] * 6
            + [pltpu.VMEM((B, FW, L), jnp.float32)] * 4
            + [pltpu.VMEM((RC, FW, L), jnp.float32)] * 2
            + [pltpu.VMEM((L,), jnp.float32)] * 2
            + [pltpu.SemaphoreType.DMA] * 16
        ),
        compiler_params=pltpu.CompilerParams(use_tc_tiling_on_sc=False),
    )
    return f(xs, sd, w_b, dtv, omv, zro, zrd)


def kernel(x, edge_index, edge_weight, diffusion_time):
    t = lax.stop_gradient(jnp.maximum(diffusion_time, 1e-8))
    dt = (t / K).astype(jnp.float32)

    src = edge_index[0]
    dst = edge_index[1]
    pad = EP - E
    fill = (jnp.arange(pad, dtype=jnp.int32) * 97) % N
    src_p = jnp.concatenate([src, fill])
    dst_p = jnp.concatenate([dst, fill])
    w_p = jnp.concatenate([edge_weight, jnp.zeros((pad,), jnp.float32)])
    w_b = jnp.broadcast_to(w_p[:, None], (EP, L))
    src_r = src_p.reshape(NS, TB, 1, B)
    dst_r = dst_p.reshape(NS, TB, 1, B)
    sd = jnp.stack([
        jnp.concatenate([src_r, dst_r], axis=2),
        jnp.concatenate([src_r + NP, dst_r], axis=2),
    ]).reshape(-1)

    xs = x.reshape(N, NC, FW, L).transpose(1, 0, 2, 3)
    xs = jnp.pad(xs, ((0, 0), (0, NP - N), (0, 0), (0, 0)))
    xs = xs.reshape(NC * NP, FW, L)
    dtv = jnp.full((L,), dt, jnp.float32)
    omv = jnp.full((L,), 1.0 - dt, jnp.float32)
    zro = jnp.zeros((RC, FW, L), jnp.float32)
    zrd = jnp.zeros((RC, L), jnp.float32)

    out, _ = _diffuse(xs, sd, w_b, dtv, omv, zro, zrd)
    out = out.reshape(NC, NP, FW, L)[:, :N]
    return out.transpose(1, 0, 2, 3).reshape(N, D)


# speedup vs baseline: 1.0108x; 1.0108x over previous
import jax
import jax.numpy as jnp
from jax import lax
from jax.experimental import pallas as pl
from jax.experimental.pallas import tpu as pltpu
from jax.experimental.pallas import tpu_sc as plsc

N = 10000
NP = 10240
E = 320000
D = 128
K = 8
NC = 2
NS = 16
L = 16
DH = D // NC
FW = DH // L
B = 128
TB = 160
TE = TB * B
EP = NS * TE
RPT = NP // NS
RC = 128
NRC = RPT // RC
NR = 4


def _body(xs, srcr, dstr, wbh, dtr, omr, zro, zrd, out, wnb,
          agg_sh, deg_sh,
          is0, is1, is2, is3, id0, id1, id2, id3,
          wb0, wb1, wb2, wb3, wrow0, wrow1,
          rw0, rw1, rw2, rw3,
          uh, ua, dtv, omv,
          sg0, sg1, sg2, sg3, ss0, ss1, ss2, ss3,
          sw0, sw1, sw2, sw3, si0, si1, si2, si3):
    cid = lax.axis_index("c")
    sid = lax.axis_index("s")
    hbase = cid * NP
    ebase = sid * TE
    sbase = cid * EP + ebase

    idx_s = (is0, is1, is2, is3)
    idx_d = (id0, id1, id2, id3)
    wbr = (wb0, wb1, wb2, wb3)
    wrow = (wrow0, wrow1)
    rows = (rw0, rw1, rw2, rw3)
    sg = (sg0, sg1, sg2, sg3)
    ss = (ss0, ss1, ss2, ss3)
    sw = (sw0, sw1, sw2, sw3)
    si = (si0, si1, si2, si3)

    pltpu.sync_copy(dtr, dtv)
    pltpu.sync_copy(omr, omv)

    for ch in range(NRC):
        r0 = sid * RPT + ch * RC
        pltpu.sync_copy(zro, agg_sh.at[pl.ds(r0, RC)])
        pltpu.sync_copy(zrd, deg_sh.at[pl.ds(r0, RC)])
        pltpu.sync_copy(xs.at[pl.ds(hbase + r0, RC)], uh)
        pltpu.sync_copy(uh, out.at[pl.ds(hbase + r0, RC)])

    plsc.subcore_barrier()

    def _dd_start(b1, r):
        pltpu.async_copy(dstr.at[pl.ds(ebase + b1 * B, B)], idx_d[r], si[r])

    def _dd_wait(b1, r):
        pltpu.make_async_copy(
            dstr.at[pl.ds(ebase + b1 * B, B)], idx_d[r], si[r]).wait()

    def _ds_start(b1, r):
        pltpu.async_copy(srcr.at[pl.ds(sbase + b1 * B, B)], idx_s[r], si[r])

    def _ds_wait(b1, r):
        pltpu.make_async_copy(
            srcr.at[pl.ds(sbase + b1 * B, B)], idx_s[r], si[r]).wait()

    def _w_start(b1, r):
        pltpu.async_copy(wbh.at[pl.ds(ebase + b1 * B, B)], wbr[r], sw[r])

    def _w_wait(b1, r):
        pltpu.make_async_copy(
            wbh.at[pl.ds(ebase + b1 * B, B)], wbr[r], sw[r]).wait()

    def _wn_start(b1, r):
        pltpu.async_copy(wnb.at[pl.ds(ebase + b1 * B, B)], wbr[r], sw[r])

    def _wn_wait(b1, r):
        pltpu.make_async_copy(
            wnb.at[pl.ds(ebase + b1 * B, B)], wbr[r], sw[r]).wait()

    _dd_start(0, 0)
    _w_start(0, 0)

    def _degA(j, c):
        for p in (0, 1):
            i = 2 * j + p
            q = 1 - p

            @pl.when(i + 1 < TB)
            def _():
                @pl.when(i >= 1)
                def _():
                    pltpu.make_async_copy(
                        wbr[q], deg_sh.at[idx_d[q]], ss[q]).wait()
                _dd_start(i + 1, q)
                _w_start(i + 1, q)

            _dd_wait(i, p)
            _w_wait(i, p)
            pltpu.async_copy(wbr[p], deg_sh.at[idx_d[p]], ss[p], add=True)
        return c
    lax.fori_loop(0, TB // 2, _degA, 0)
    pltpu.make_async_copy(wbr[0], deg_sh.at[idx_d[0]], ss[0]).wait()
    pltpu.make_async_copy(wbr[1], deg_sh.at[idx_d[1]], ss[1]).wait()

    plsc.subcore_barrier()

    def _deg_start(p):
        pltpu.async_copy(deg_sh.at[idx_d[p]], wrow[p], sg[p])

    def _deg_wait(p):
        pltpu.make_async_copy(deg_sh.at[idx_d[p]], wrow[p], sg[p]).wait()

    _dd_start(0, 0)
    _dd_wait(0, 0)
    _w_start(0, 0)
    _deg_start(0)

    def _wnB(j, c):
        for p in (0, 1):
            i = 2 * j + p
            q = 1 - p

            @pl.when(i + 1 < TB)
            def _():
                @pl.when(i >= 1)
                def _():
                    pltpu.make_async_copy(
                        wbr[q], wnb.at[pl.ds(ebase + (i - 1) * B, B)],
                        sg[2 + q]).wait()
                _dd_start(i + 1, q)
                _w_start(i + 1, q)
                _dd_wait(i + 1, q)
                _deg_start(q)

            _w_wait(i, p)
            _deg_wait(p)

            def _edge(e2, c2):
                dv = dtv[...]
                for u in range(2):
                    e = e2 * 2 + u
                    wbr[p][e] = dv * wbr[p][e] / (wrow[p][e] + 1e-8)
                return c2
            lax.fori_loop(0, B // 2, _edge, 0)

            pltpu.async_copy(
                wbr[p], wnb.at[pl.ds(ebase + i * B, B)], sg[2 + p])
        return c
    lax.fori_loop(0, TB // 2, _wnB, 0)
    pltpu.make_async_copy(
        wbr[0], wnb.at[pl.ds(ebase + (TB - 2) * B, B)], sg[2]).wait()
    pltpu.make_async_copy(
        wbr[1], wnb.at[pl.ds(ebase + (TB - 1) * B, B)], sg[3]).wait()

    plsc.subcore_barrier()

    def _gather_start(b1, r):
        pltpu.async_copy(out.at[idx_s[r]], rows[r], sg[r])

    def _gather_wait(b1, r):
        pltpu.make_async_copy(out.at[idx_s[r]], rows[r], sg[r]).wait()

    def _scat_start(b1, r):
        pltpu.async_copy(rows[r], agg_sh.at[idx_d[r]], ss[r], add=True)

    def _scat_wait(b1, r):
        pltpu.make_async_copy(rows[r], agg_sh.at[idx_d[r]], ss[r]).wait()

    def _di_start(b1, r):
        _ds_start(b1, r)
        _dd_start(b1, r)

    def _di_wait(b1, r):
        _ds_wait(b1, r)
        _dd_wait(b1, r)

    def _step(k, c):
        _di_start(0, 0)
        _di_wait(0, 0)
        _wn_start(0, 0)
        _gather_start(0, 0)
        _di_start(1, 1)

        def _quad(j, c2):
            for r in range(NR):
                i = NR * j + r
                r1 = (r + 1) % NR
                r2 = (r + 2) % NR

                @pl.when(i + 2 < TB)
                def _():
                    @pl.when(i >= 2)
                    def _():
                        _scat_wait(i - 2, r2)
                    _di_start(i + 2, r2)

                @pl.when(i + 1 < TB)
                def _():
                    _di_wait(i + 1, r1)
                    _wn_start(i + 1, r1)
                    _gather_start(i + 1, r1)

                _gather_wait(i, r)
                _wn_wait(i, r)

                def _edge(e4, c3):
                    for u in range(4):
                        e = e4 * 4 + u
                        wv = wbr[r][e]
                        for f in range(FW):
                            rows[r][e, f] = rows[r][e, f] * wv
                    return c3
                lax.fori_loop(0, B // 4, _edge, 0)

                _scat_start(i, r)
            return c2
        lax.fori_loop(0, TB // NR, _quad, 0)
        _scat_wait(TB - 4, 0)
        _scat_wait(TB - 3, 1)
        _scat_wait(TB - 2, 2)
        _scat_wait(TB - 1, 3)

        plsc.subcore_barrier()

        for ch in range(NRC):
            r0l = sid * RPT + ch * RC
            r0g = hbase + r0l
            pltpu.sync_copy(out.at[pl.ds(r0g, RC)], uh)
            pltpu.sync_copy(agg_sh.at[pl.ds(r0l, RC)], ua)

            def _upd(r2c, c2):
                om = omv[...]
                for u in range(2):
                    rr = r2c * 2 + u
                    for f in range(FW):
                        uh[rr, f] = om * uh[rr, f] + ua[rr, f]
                return c2
            lax.fori_loop(0, RC // 2, _upd, 0)

            pltpu.sync_copy(uh, out.at[pl.ds(r0g, RC)])
            pltpu.sync_copy(zro, agg_sh.at[pl.ds(r0l, RC)])

        plsc.subcore_barrier()
        return c
    lax.fori_loop(0, K, _step, 0)


@jax.jit
def _diffuse(xs, src2, dst, w_b, dtv, omv, zro, zrd):
    mesh = plsc.VectorSubcoreMesh(core_axis_name="c", subcore_axis_name="s",
                                  num_cores=NC, num_subcores=NS)
    f = pl.kernel(
        _body,
        out_type=(
            jax.ShapeDtypeStruct((NC * NP, FW, L), jnp.float32),
            jax.ShapeDtypeStruct((EP, L), jnp.float32),
        ),
        mesh=mesh,
        scratch_types=(
            [pltpu.VMEM_SHARED((NP, FW, L), jnp.float32),
             pltpu.VMEM_SHARED((NP, L), jnp.float32)]
            + [pltpu.VMEM((B,), jnp.int32)] * 8
            + [pltpu.VMEM((B, L), jnp.float32)] * 6
            + [pltpu.VMEM((B, FW, L), jnp.float32)] * 4
            + [pltpu.VMEM((RC, FW, L), jnp.float32)] * 2
            + [pltpu.VMEM((L,), jnp.float32)] * 2
            + [pltpu.SemaphoreType.DMA] * 16
        ),
        compiler_params=pltpu.CompilerParams(use_tc_tiling_on_sc=False),
    )
    return f(xs, src2, dst, w_b, dtv, omv, zro, zrd)


def kernel(x, edge_index, edge_weight, diffusion_time):
    t = lax.stop_gradient(jnp.maximum(diffusion_time, 1e-8))
    dt = (t / K).astype(jnp.float32)

    src = edge_index[0]
    dst = edge_index[1]
    pad = EP - E
    fill = (jnp.arange(pad, dtype=jnp.int32) * 97) % N
    src_p = jnp.concatenate([src, fill])
    dst_p = jnp.concatenate([dst, fill])
    w_p = jnp.concatenate([edge_weight, jnp.zeros((pad,), jnp.float32)])
    w_b = jnp.broadcast_to(w_p[:, None], (EP, L))
    src2 = jnp.concatenate([src_p, src_p + NP])

    xs = x.reshape(N, NC, FW, L).transpose(1, 0, 2, 3)
    xs = jnp.pad(xs, ((0, 0), (0, NP - N), (0, 0), (0, 0)))
    xs = xs.reshape(NC * NP, FW, L)
    dtv = jnp.full((L,), dt, jnp.float32)
    omv = jnp.full((L,), 1.0 - dt, jnp.float32)
    zro = jnp.zeros((RC, FW, L), jnp.float32)
    zrd = jnp.zeros((RC, L), jnp.float32)

    out, _ = _diffuse(xs, src2, dst_p, w_b, dtv, omv, zro, zrd)
    out = out.reshape(NC, NP, FW, L)[:, :N]
    return out.transpose(1, 0, 2, 3).reshape(N, D)
